# Initial kernel scaffold; baseline (speedup 1.0000x reference)
import jax
import jax.numpy as jnp
from jax.experimental import pallas as pl


def kernel(x, edge_index, W1, b1, W2, b2, Wfc, bfc):
    raise NotImplementedError("write your pallas kernel here")


# speedup vs baseline: 13.1098x; 13.1098x over previous
import functools

import jax
import jax.numpy as jnp
from jax import lax
from jax.experimental import pallas as pl
from jax.experimental.pallas import tpu as pltpu
from jax.experimental.pallas import tpu_sc as plsc

N_NODES = 10000
N_EDGES = 160000
D_IN = 256
D_H1 = 64
D_H2 = 32

N_PAD = 10240
NW = 32
GROUPS = 40
E_PAD = NW * GROUPS * 128
E_TILE = GROUPS * 128
ROWS_PER_TILE = N_PAD // 16

_mesh = plsc.VectorSubcoreMesh(core_axis_name="c", subcore_axis_name="s")
_sc_params = pltpu.CompilerParams(
    needs_layout_passes=False, use_tc_tiling_on_sc=False
)


@functools.partial(
    pl.kernel,
    out_type=jax.ShapeDtypeStruct((NW, N_PAD), jnp.float32),
    mesh=_mesh,
    compiler_params=_sc_params,
    scratch_types=[
        pltpu.VMEM((E_TILE,), jnp.int32),
        pltpu.VMEM((N_PAD,), jnp.float32),
    ],
)
def _deg_kernel(dst_hbm, out_hbm, idx_v, deg_v):
    c = lax.axis_index("c")
    s = lax.axis_index("s")
    wid = s * 2 + c

    zero16 = jnp.zeros((16,), jnp.float32)

    def zero_body(i, _):
        deg_v[pl.ds(i * 16, 16)] = zero16
        return 0

    lax.fori_loop(0, N_PAD // 16, zero_body, 0)

    pltpu.sync_copy(dst_hbm.at[wid], idx_v)

    ones16 = jnp.ones((16,), jnp.float32)

    def edge_body(i, _):
        idx16 = idx_v[pl.ds(i * 16, 16)]
        plsc.addupdate_scatter(deg_v, [idx16], ones16)
        return 0

    lax.fori_loop(0, E_TILE // 16, edge_body, 0)

    pltpu.sync_copy(deg_v, out_hbm.at[wid])


def _make_mp_kernel(D):
    @functools.partial(
        pl.kernel,
        out_type=jax.ShapeDtypeStruct((2, N_PAD, D), jnp.float32),
        mesh=_mesh,
        compiler_params=_sc_params,
        scratch_types=[
            pltpu.VMEM((GROUPS, 128), jnp.int32),
            pltpu.VMEM((GROUPS, 128), jnp.int32),
            pltpu.VMEM((128, D), jnp.float32),
            pltpu.VMEM_SHARED((N_PAD, D), jnp.float32),
            pltpu.SemaphoreType.DMA,
        ],
    )
    def mp(g_hbm, src_hbm, dst_hbm, out_hbm, src_v, dst_v, rows_v, acc_sh, sem):
        c = lax.axis_index("c")
        s = lax.axis_index("s")
        wid = s * 2 + c

        zero16 = jnp.zeros((16,), jnp.float32)
        dv = D // 16

        def zero_body(i, _):
            rows_v[i // dv, pl.ds((i % dv) * 16, 16)] = zero16
            return 0

        lax.fori_loop(0, 128 * dv, zero_body, 0)

        base_row = s * ROWS_PER_TILE
        for t in range(ROWS_PER_TILE // 128):
            pltpu.sync_copy(rows_v, acc_sh.at[pl.ds(base_row + t * 128, 128)])

        pltpu.sync_copy(src_hbm.at[wid], src_v)
        pltpu.sync_copy(dst_hbm.at[wid], dst_v)
        plsc.subcore_barrier()

        def group_body(j, _):
            pltpu.async_copy(g_hbm.at[src_v.at[j]], rows_v, sem).wait()
            pltpu.sync_copy(rows_v, acc_sh.at[dst_v.at[j]], add=True)
            return 0

        lax.fori_loop(0, GROUPS, group_body, 0)

        plsc.subcore_barrier()

        for t in range(ROWS_PER_TILE // 128):
            r = base_row + t * 128
            pltpu.sync_copy(acc_sh.at[pl.ds(r, 128)], rows_v)
            pltpu.sync_copy(rows_v, out_hbm.at[c, pl.ds(r, 128)])

    return mp


_mp_kernel_64 = _make_mp_kernel(D_H1)
_mp_kernel_32 = _make_mp_kernel(D_H2)


def _deg_reduce_body(p_ref, dis_ref, inv_ref):
    deg = jnp.sum(p_ref[...], axis=0, keepdims=True) + 1.0
    dis_ref[...] = lax.rsqrt(deg)
    inv_ref[...] = 1.0 / deg


def _layer1_body(x_ref, w_ref, dis_ref, h_ref, g_ref):
    h = jnp.dot(x_ref[...], w_ref[...], preferred_element_type=jnp.float32)
    h_ref[...] = h
    g_ref[...] = h * dis_ref[...]


def _layer2_body(h1_ref, s_ref, dis_ref, inv_ref, b1_ref, w2_ref, h2_ref, g2_ref):
    agg = dis_ref[...] * (s_ref[0] + s_ref[1]) + h1_ref[...] * inv_ref[...] + b1_ref[...]
    a1 = jnp.maximum(agg, 0.0)
    h2 = jnp.dot(a1, w2_ref[...], preferred_element_type=jnp.float32)
    h2_ref[...] = h2
    g2_ref[...] = h2 * dis_ref[...]


def _head_body(h2_ref, s_ref, dis_ref, inv_ref, b2_ref, wfc_ref, bfc_ref, o_ref):
    agg = dis_ref[...] * (s_ref[0] + s_ref[1]) + h2_ref[...] * inv_ref[...] + b2_ref[...]
    a2 = jnp.maximum(agg, 0.0)
    o_ref[...] = (
        jnp.dot(a2, wfc_ref[...], preferred_element_type=jnp.float32) + bfc_ref[...]
    )


_BM = 1280
_NBLK = N_PAD // _BM


def _rows_spec(width):
    return pl.BlockSpec((_BM, width), lambda i: (i, 0))


def _full_spec(shape):
    return pl.BlockSpec(shape, lambda i: tuple(0 for _ in shape))


def _partials_spec(width):
    return pl.BlockSpec((2, _BM, width), lambda i: (0, i, 0))


def kernel(x, edge_index, W1, b1, W2, b2, Wfc, bfc):
    f32 = jnp.float32
    xp = jnp.zeros((N_PAD, D_IN), f32).at[:N_NODES].set(x)
    pad = jnp.full((E_PAD - N_EDGES,), N_NODES, jnp.int32)
    src = jnp.concatenate([edge_index[0].astype(jnp.int32), pad]).reshape(NW, GROUPS, 128)
    dst = jnp.concatenate([edge_index[1].astype(jnp.int32), pad]).reshape(NW, GROUPS, 128)
    dst_flat = dst.reshape(NW, E_TILE)

    deg_partials = _deg_kernel(dst_flat)

    dis_row, inv_row = pl.pallas_call(
        _deg_reduce_body,
        grid=(1,),
        in_specs=[_full_spec((NW, N_PAD))],
        out_specs=[_full_spec((1, N_PAD)), _full_spec((1, N_PAD))],
        out_shape=[jax.ShapeDtypeStruct((1, N_PAD), f32)] * 2,
    )(deg_partials)
    dis = dis_row.reshape(N_PAD, 1)
    inv = inv_row.reshape(N_PAD, 1)

    h1, g1 = pl.pallas_call(
        _layer1_body,
        grid=(_NBLK,),
        in_specs=[
            _rows_spec(D_IN),
            _full_spec((D_IN, D_H1)),
            _rows_spec(1),
        ],
        out_specs=[_rows_spec(D_H1), _rows_spec(D_H1)],
        out_shape=[jax.ShapeDtypeStruct((N_PAD, D_H1), f32)] * 2,
    )(xp, W1, dis)

    s1 = _mp_kernel_64(g1, src, dst)

    h2, g2 = pl.pallas_call(
        _layer2_body,
        grid=(_NBLK,),
        in_specs=[
            _rows_spec(D_H1),
            _partials_spec(D_H1),
            _rows_spec(1),
            _rows_spec(1),
            _full_spec((1, D_H1)),
            _full_spec((D_H1, D_H2)),
        ],
        out_specs=[_rows_spec(D_H2), _rows_spec(D_H2)],
        out_shape=[jax.ShapeDtypeStruct((N_PAD, D_H2), f32)] * 2,
    )(h1, s1, dis, inv, b1.reshape(1, D_H1), W2)

    s2 = _mp_kernel_32(g2, src, dst)

    o = pl.pallas_call(
        _head_body,
        grid=(_NBLK,),
        in_specs=[
            _rows_spec(D_H2),
            _partials_spec(D_H2),
            _rows_spec(1),
            _rows_spec(1),
            _full_spec((1, D_H2)),
            _full_spec((D_H2, 1)),
            _full_spec((1, 1)),
        ],
        out_specs=_rows_spec(1),
        out_shape=jax.ShapeDtypeStruct((N_PAD, 1), f32),
    )(h2, s2, dis, inv, b2.reshape(1, D_H2), Wfc, bfc.reshape(1, 1))

    return o[:N_NODES, 0]


# speedup vs baseline: 14.5413x; 1.1092x over previous
import functools

import jax
import jax.numpy as jnp
from jax import lax
from jax.experimental import pallas as pl
from jax.experimental.pallas import tpu as pltpu
from jax.experimental.pallas import tpu_sc as plsc

N_NODES = 10000
N_EDGES = 160000
D_IN = 256
D_H1 = 64
D_H2 = 32

N_PAD = 10240
NW = 32
GROUPS = 40
E_PAD = NW * GROUPS * 128
E_TILE = GROUPS * 128
ROWS_PER_TILE = N_PAD // 16

_mesh = plsc.VectorSubcoreMesh(core_axis_name="c", subcore_axis_name="s")
_sc_params = pltpu.CompilerParams(
    needs_layout_passes=False, use_tc_tiling_on_sc=False
)


@functools.partial(
    pl.kernel,
    out_type=jax.ShapeDtypeStruct((NW, N_PAD), jnp.float32),
    mesh=_mesh,
    compiler_params=_sc_params,
    scratch_types=[
        pltpu.VMEM((E_TILE,), jnp.int32),
        pltpu.VMEM((N_PAD,), jnp.float32),
    ],
)
def _deg_kernel(dst_hbm, out_hbm, idx_v, deg_v):
    c = lax.axis_index("c")
    s = lax.axis_index("s")
    wid = s * 2 + c

    zero16 = jnp.zeros((16,), jnp.float32)

    def zero_body(i, _):
        deg_v[pl.ds(i * 16, 16)] = zero16
        return 0

    lax.fori_loop(0, N_PAD // 16, zero_body, 0)

    pltpu.sync_copy(dst_hbm.at[wid], idx_v)

    ones16 = jnp.ones((16,), jnp.float32)

    def edge_body(i, _):
        idx16 = idx_v[pl.ds(i * 16, 16)]
        plsc.addupdate_scatter(deg_v, [idx16], ones16)
        return 0

    lax.fori_loop(0, E_TILE // 16, edge_body, 0)

    pltpu.sync_copy(deg_v, out_hbm.at[wid])


def _make_mp_kernel(D):
    @functools.partial(
        pl.kernel,
        out_type=jax.ShapeDtypeStruct((2, N_PAD, D), jnp.float32),
        mesh=_mesh,
        compiler_params=_sc_params,
        scratch_types=[
            pltpu.VMEM((GROUPS, 128), jnp.int32),
            pltpu.VMEM((GROUPS, 128), jnp.int32),
            pltpu.VMEM((128, D), jnp.float32),
            pltpu.VMEM((128, D), jnp.float32),
            pltpu.VMEM_SHARED((N_PAD, D), jnp.float32),
            pltpu.SemaphoreType.DMA,
            pltpu.SemaphoreType.DMA,
        ],
    )
    def mp(g_hbm, src_hbm, dst_hbm, out_hbm, src_v, dst_v, rows_a, rows_b,
           acc_sh, sem_a, sem_b):
        c = lax.axis_index("c")
        s = lax.axis_index("s")
        wid = s * 2 + c

        zero16 = jnp.zeros((16,), jnp.float32)
        dv = D // 16

        def zero_body(i, _):
            rows_a[i // dv, pl.ds((i % dv) * 16, 16)] = zero16
            return 0

        lax.fori_loop(0, 128 * dv, zero_body, 0)

        base_row = s * ROWS_PER_TILE
        for t in range(ROWS_PER_TILE // 128):
            pltpu.sync_copy(rows_a, acc_sh.at[pl.ds(base_row + t * 128, 128)])

        pltpu.sync_copy(src_hbm.at[wid], src_v)
        pltpu.sync_copy(dst_hbm.at[wid], dst_v)
        plsc.subcore_barrier()

        pltpu.async_copy(g_hbm.at[src_v.at[0]], rows_a, sem_a)

        def group_body(i, _):
            j0 = 2 * i
            pltpu.async_copy(g_hbm.at[src_v.at[j0 + 1]], rows_b, sem_b)
            pltpu.make_async_copy(g_hbm.at[src_v.at[j0]], rows_a, sem_a).wait()
            pltpu.sync_copy(rows_a, acc_sh.at[dst_v.at[j0]], add=True)
            nxt = jnp.minimum(j0 + 2, GROUPS - 2)
            pltpu.async_copy(g_hbm.at[src_v.at[nxt]], rows_a, sem_a)
            pltpu.make_async_copy(g_hbm.at[src_v.at[j0 + 1]], rows_b, sem_b).wait()
            pltpu.sync_copy(rows_b, acc_sh.at[dst_v.at[j0 + 1]], add=True)
            return 0

        lax.fori_loop(0, GROUPS // 2, group_body, 0)
        pltpu.make_async_copy(g_hbm.at[src_v.at[0]], rows_a, sem_a).wait()

        plsc.subcore_barrier()

        for t in range(ROWS_PER_TILE // 128):
            r = base_row + t * 128
            pltpu.sync_copy(acc_sh.at[pl.ds(r, 128)], rows_a)
            pltpu.sync_copy(rows_a, out_hbm.at[c, pl.ds(r, 128)])

    return mp


_mp_kernel_64 = _make_mp_kernel(D_H1)
_mp_kernel_32 = _make_mp_kernel(D_H2)


def _deg_reduce_body(p_ref, dis_ref, inv_ref):
    deg = jnp.sum(p_ref[...], axis=0, keepdims=True) + 1.0
    dis_ref[...] = lax.rsqrt(deg)
    inv_ref[...] = 1.0 / deg


def _layer1_body(x_ref, w_ref, dis_ref, h_ref, g_ref):
    h = jnp.dot(x_ref[...], w_ref[...], preferred_element_type=jnp.float32)
    h_ref[...] = h
    g_ref[...] = h * dis_ref[...]


def _layer2_body(h1_ref, s_ref, dis_ref, inv_ref, b1_ref, w2_ref, h2_ref, g2_ref):
    agg = dis_ref[...] * (s_ref[0] + s_ref[1]) + h1_ref[...] * inv_ref[...] + b1_ref[...]
    a1 = jnp.maximum(agg, 0.0)
    h2 = jnp.dot(a1, w2_ref[...], preferred_element_type=jnp.float32)
    h2_ref[...] = h2
    g2_ref[...] = h2 * dis_ref[...]


def _head_body(h2_ref, s_ref, dis_ref, inv_ref, b2_ref, wfc_ref, bfc_ref, o_ref):
    agg = dis_ref[...] * (s_ref[0] + s_ref[1]) + h2_ref[...] * inv_ref[...] + b2_ref[...]
    a2 = jnp.maximum(agg, 0.0)
    o_ref[...] = (
        jnp.dot(a2, wfc_ref[...], preferred_element_type=jnp.float32) + bfc_ref[...]
    )


_BM = 1280
_NBLK = N_PAD // _BM


def _rows_spec(width):
    return pl.BlockSpec((_BM, width), lambda i: (i, 0))


def _full_spec(shape):
    return pl.BlockSpec(shape, lambda i: tuple(0 for _ in shape))


def _partials_spec(width):
    return pl.BlockSpec((2, _BM, width), lambda i: (0, i, 0))


def kernel(x, edge_index, W1, b1, W2, b2, Wfc, bfc):
    f32 = jnp.float32
    xp = jnp.zeros((N_PAD, D_IN), f32).at[:N_NODES].set(x)
    pad = jnp.full((E_PAD - N_EDGES,), N_NODES, jnp.int32)
    src = jnp.concatenate([edge_index[0].astype(jnp.int32), pad]).reshape(NW, GROUPS, 128)
    dst = jnp.concatenate([edge_index[1].astype(jnp.int32), pad]).reshape(NW, GROUPS, 128)
    dst_flat = dst.reshape(NW, E_TILE)

    deg_partials = _deg_kernel(dst_flat)

    dis_row, inv_row = pl.pallas_call(
        _deg_reduce_body,
        grid=(1,),
        in_specs=[_full_spec((NW, N_PAD))],
        out_specs=[_full_spec((1, N_PAD)), _full_spec((1, N_PAD))],
        out_shape=[jax.ShapeDtypeStruct((1, N_PAD), f32)] * 2,
    )(deg_partials)
    dis = dis_row.reshape(N_PAD, 1)
    inv = inv_row.reshape(N_PAD, 1)

    h1, g1 = pl.pallas_call(
        _layer1_body,
        grid=(_NBLK,),
        in_specs=[
            _rows_spec(D_IN),
            _full_spec((D_IN, D_H1)),
            _rows_spec(1),
        ],
        out_specs=[_rows_spec(D_H1), _rows_spec(D_H1)],
        out_shape=[jax.ShapeDtypeStruct((N_PAD, D_H1), f32)] * 2,
    )(xp, W1, dis)

    s1 = _mp_kernel_64(g1, src, dst)

    h2, g2 = pl.pallas_call(
        _layer2_body,
        grid=(_NBLK,),
        in_specs=[
            _rows_spec(D_H1),
            _partials_spec(D_H1),
            _rows_spec(1),
            _rows_spec(1),
            _full_spec((1, D_H1)),
            _full_spec((D_H1, D_H2)),
        ],
        out_specs=[_rows_spec(D_H2), _rows_spec(D_H2)],
        out_shape=[jax.ShapeDtypeStruct((N_PAD, D_H2), f32)] * 2,
    )(h1, s1, dis, inv, b1.reshape(1, D_H1), W2)

    s2 = _mp_kernel_32(g2, src, dst)

    o = pl.pallas_call(
        _head_body,
        grid=(_NBLK,),
        in_specs=[
            _rows_spec(D_H2),
            _partials_spec(D_H2),
            _rows_spec(1),
            _rows_spec(1),
            _full_spec((1, D_H2)),
            _full_spec((D_H2, 1)),
            _full_spec((1, 1)),
        ],
        out_specs=_rows_spec(1),
        out_shape=jax.ShapeDtypeStruct((N_PAD, 1), f32),
    )(h2, s2, dis, inv, b2.reshape(1, D_H2), Wfc, bfc.reshape(1, 1))

    return o[:N_NODES, 0]
